# baseline (speedup 1.0000x reference)
import functools

import jax
import jax.numpy as jnp
from jax import lax
from jax.experimental import pallas as pl
from jax.experimental.pallas import tpu as pltpu
from jax.experimental.pallas import tpu_sc as plsc

IN_DIM = 50000
H = 64
NUM_RELS = 256
SEQ_LEN = 10
E = 800000
ES = E // SEQ_LEN

NC = 2
NS = 16
L = 16

EPT = ES // NS
K = 128
EPT_PAD = 5120
NB = EPT_PAD // K

QTR = 10752
NSUB = 3
TPT = QTR // NS
DUMP = QTR
AGG_ROWS = QTR + 8
NCH = 56
NPAD = 57344

_f32 = jnp.float32


def _sc_aggregate(srcp, dstp, rtp, ent_pad, wrel_flat, xself):
  mesh = plsc.VectorSubcoreMesh(core_axis_name="c", subcore_axis_name="s",
                                num_cores=NC, num_subcores=NS)

  @functools.partial(
      pl.kernel,
      out_type=jax.ShapeDtypeStruct((NC, SEQ_LEN, H), _f32),
      mesh=mesh,
      scratch_types=[
          pltpu.VMEM((EPT_PAD,), jnp.int32),
          pltpu.VMEM((EPT_PAD,), jnp.int32),
          pltpu.VMEM((EPT_PAD,), jnp.int32),
          pltpu.VMEM((EPT_PAD + L,), jnp.int32),
          pltpu.VMEM((EPT_PAD + L,), jnp.int32),
          pltpu.VMEM((EPT_PAD + L,), jnp.int32),
          pltpu.VMEM((NB, K), jnp.int32),
          pltpu.VMEM((NUM_RELS * H,), _f32),
          pltpu.VMEM((K, H), _f32),
          pltpu.VMEM((K, L), _f32),
          pltpu.VMEM((NCH, H), _f32),
          pltpu.VMEM((NCH, L), _f32),
          pltpu.VMEM((NCH, H), _f32),
          pltpu.VMEM((NCH, H), _f32),
          pltpu.VMEM((NCH, L), _f32),
          pltpu.VMEM((NS, H), _f32),
          pltpu.VMEM((1, H), _f32),
          pltpu.VMEM_SHARED((AGG_ROWS, H), _f32),
          pltpu.VMEM_SHARED((AGG_ROWS, L), _f32),
          pltpu.VMEM_SHARED((NS, H), _f32),
          pltpu.SemaphoreType.DMA,
      ],
      compiler_params=pltpu.CompilerParams(use_tc_tiling_on_sc=False),
  )
  def body(src_h, dst_h, rt_h, ent_h, wrel_h, xself_h, out_h,
           src_v, rt_v, dstraw_v, csrc_v, crt_v, cdst_v, dst2_v,
           wrel_v, rows_v, ones_v,
           agg_v, deg_v, xs_v, zero_v, zerod_v, stmax_v, mymax_v,
           agg_sh, deg_sh, stage_sh, sem):
    c = lax.axis_index("c")
    t = lax.axis_index("s")

    pltpu.sync_copy(wrel_h, wrel_v)
    one16 = jnp.full((L,), 1.0, dtype=_f32)
    z16 = jnp.zeros((L,), dtype=_f32)

    def fill_ones(n, _):
      ones_v[n, pl.ds(0, L)] = one16
      return 0
    lax.fori_loop(0, K, fill_ones, 0)

    def fill_zero(n, _):
      for k in range(H // L):
        zero_v[n, pl.ds(k * L, L)] = z16
      zerod_v[n, pl.ds(0, L)] = z16
      return 0
    lax.fori_loop(0, NCH, fill_zero, 0)

    def snapshot(s, _):
      off = (s * NS + t) * EPT_PAD
      pltpu.sync_copy(src_h.at[pl.ds(off, EPT_PAD)], src_v)
      pltpu.sync_copy(rt_h.at[pl.ds(off, EPT_PAD)], rt_v)
      pltpu.sync_copy(dst_h.at[pl.ds(off, EPT_PAD)], dstraw_v)

      def subrange(sub, mxall):
        lo = c * 25000 + sub * QTR
        hi = jnp.minimum(lo + QTR, IN_DIM)

        def zchunk(i, _):
          r = t * TPT + i * NCH
          pltpu.sync_copy(zero_v, agg_sh.at[pl.ds(r, NCH)])
          pltpu.sync_copy(zerod_v, deg_sh.at[pl.ds(r, NCH)])
          return 0
        lax.fori_loop(0, TPT // NCH, zchunk, 0)

        dumps = jnp.full((L,), DUMP, dtype=jnp.int32)
        zi = jnp.zeros((L,), dtype=jnp.int32)

        def prefill(j, _):
          csrc_v[pl.ds(j * L, L)] = zi
          crt_v[pl.ds(j * L, L)] = zi
          cdst_v[pl.ds(j * L, L)] = dumps
          return 0
        lax.fori_loop(0, (EPT_PAD + L) // L, prefill, 0)

        def cgroup(j, cnt):
          d = dstraw_v[pl.ds(j * L, L)]
          inr = (d >= lo) & (d < hi)
          csrc_v[pl.ds(j * L, L)] = src_v[pl.ds(j * L, L)]
          crt_v[pl.ds(j * L, L)] = rt_v[pl.ds(j * L, L)]
          cdst_v[pl.ds(j * L, L)] = jnp.where(inr, d - lo, DUMP)
          return cnt + L
        cnt = lax.fori_loop(0, EPT_PAD // L, cgroup, 0)
        plsc.subcore_barrier()

        def eblock(b, _):
          def cpy(jj, _):
            dst2_v[b, pl.ds(jj * L, L)] = cdst_v[pl.ds(b * K + jj * L, L)]
            return 0
          lax.fori_loop(0, K // L, cpy, 0)

          pltpu.async_copy(ent_h.at[csrc_v.at[pl.ds(b * K, K)]], rows_v,
                           sem).wait()

          def emul(j, _):
            rt16 = crt_v[pl.ds(b * K + j * L, L)]
            for ee in range(L):
              e = j * L + ee
              wb = rt16[ee] * H
              for k in range(H // L):
                rows_v[e, pl.ds(k * L, L)] = (
                    rows_v[e, pl.ds(k * L, L)] * wrel_v[pl.ds(wb + k * L, L)])
            return 0
          lax.fori_loop(0, K // L, emul, 0)

          pltpu.sync_copy(rows_v, agg_sh.at[dst2_v.at[b]], add=True)
          pltpu.sync_copy(ones_v, deg_sh.at[dst2_v.at[b]], add=True)
          return 0
        nblk = (cnt + K - 1) // K
        lax.fori_loop(0, NB, lambda b, x: lax.cond(b < nblk, eblock, lambda b, x: 0, b, x), 0)
        plsc.subcore_barrier()

        def nchunk(i, mx):
          r = t * TPT + i * NCH
          pltpu.sync_copy(agg_sh.at[pl.ds(r, NCH)], agg_v)
          pltpu.sync_copy(deg_sh.at[pl.ds(r, NCH)], deg_v)
          pltpu.sync_copy(xself_h.at[pl.ds(lo + r, NCH)], xs_v)

          def node(n, mx):
            d16 = deg_v[n, pl.ds(0, L)]
            bc = 1.0 / jnp.maximum(d16, 1.0)
            new = []
            for k in range(H // L):
              v = agg_v[n, pl.ds(k * L, L)] * bc + xs_v[n, pl.ds(k * L, L)]
              v = jnp.maximum(v, 0.0)
              new.append(jnp.maximum(mx[k], v))
            return tuple(new)
          return lax.fori_loop(0, NCH, node, mx)

        mx = lax.fori_loop(0, TPT // NCH, nchunk, mxall)
        plsc.subcore_barrier()
        return mx

      mx0 = tuple(jnp.zeros((L,), dtype=_f32) for _ in range(H // L))
      mx = lax.fori_loop(0, NSUB, subrange, mx0)
      for k in range(H // L):
        mymax_v[0, pl.ds(k * L, L)] = mx[k]
      pltpu.sync_copy(mymax_v, stage_sh.at[pl.ds(t, 1)])
      plsc.subcore_barrier()

      @pl.when(t == 0)
      def _():
        pltpu.sync_copy(stage_sh, stmax_v)

        def tred(u, mr):
          return tuple(
              jnp.maximum(mr[k], stmax_v[u, pl.ds(k * L, L)])
              for k in range(H // L))
        mr = lax.fori_loop(0, NS, tred,
                           tuple(jnp.zeros((L,), dtype=_f32)
                                 for _ in range(H // L)))
        for k in range(H // L):
          mymax_v[0, pl.ds(k * L, L)] = mr[k]
        pltpu.sync_copy(mymax_v, out_h.at[c, pl.ds(s, 1)])
      plsc.subcore_barrier()
      return 0

    lax.fori_loop(0, SEQ_LEN, snapshot, 0)

  return body(srcp, dstp, rtp, ent_pad, wrel_flat, xself)


def _tc_xself(ent_pad, W_self):
  blk = NPAD // 8

  def body(e_ref, w_ref, o_ref):
    o_ref[...] = jnp.dot(e_ref[...], w_ref[...],
                         preferred_element_type=_f32)

  return pl.pallas_call(
      body,
      grid=(8,),
      in_specs=[
          pl.BlockSpec((blk, H), lambda i: (i, 0)),
          pl.BlockSpec((H, H), lambda i: (0, 0)),
      ],
      out_specs=pl.BlockSpec((blk, H), lambda i: (i, 0)),
      out_shape=jax.ShapeDtypeStruct((NPAD, H), _f32),
  )(ent_pad, W_self)


def _tc_gru(gs2, W_ih, W_hh, b_ih, b_hh):
  def body(g_ref, wi_ref, wh_ref, bi_ref, bh_ref, o_ref):
    gs = jnp.maximum(g_ref[0], g_ref[1])
    bi = bi_ref[...]
    bh = bh_ref[...]

    h = jnp.zeros((1, H), dtype=_f32)
    for s in range(SEQ_LEN):
      x = gs[s:s + 1]
      gi = jnp.dot(x, wi_ref[...], preferred_element_type=_f32) + bi
      gh = jnp.dot(h, wh_ref[...], preferred_element_type=_f32) + bh
      ir, iz, in_ = gi[:, :H], gi[:, H:2 * H], gi[:, 2 * H:]
      hr, hz, hn = gh[:, :H], gh[:, H:2 * H], gh[:, 2 * H:]
      r = jax.nn.sigmoid(ir + hr)
      z = jax.nn.sigmoid(iz + hz)
      n = jnp.tanh(in_ + r * hn)
      h = (1.0 - z) * n + z * h
    o_ref[...] = h

  return pl.pallas_call(
      body,
      out_shape=jax.ShapeDtypeStruct((1, H), _f32),
  )(gs2, W_ih, W_hh, b_ih.reshape(1, 3 * H), b_hh.reshape(1, 3 * H))


LPAD = 50176
LBLK = 1024


def _tc_loss(s_q, W_lin_pad, b_lin_pad, tpo_pad):
  nt = LPAD // LBLK

  def body(sq_ref, w_ref, b_ref, t_ref, o_ref, acc):
    i = pl.program_id(0)

    @pl.when(i == 0)
    def _():
      acc[0] = -1e30
      acc[1] = 0.0
      acc[2] = 0.0

    p = (jnp.dot(sq_ref[...], w_ref[...], preferred_element_type=_f32)
         + b_ref[...])
    tb = jnp.sum(t_ref[...], axis=0, keepdims=True) * (1.0 / 64.0)
    m_old = acc[0]
    m_new = jnp.maximum(m_old, jnp.max(p))
    se = acc[1] * jnp.exp(m_old - m_new) + jnp.sum(jnp.exp(p - m_new))
    acc[0] = m_new
    acc[1] = se
    acc[2] = acc[2] + jnp.sum(tb * p)

    @pl.when(i == nt - 1)
    def _():
      o_ref[...] = jnp.full((1, 1), (acc[0] + jnp.log(acc[1])) - acc[2],
                            dtype=_f32)

  return pl.pallas_call(
      body,
      grid=(nt,),
      in_specs=[
          pl.BlockSpec((1, H), lambda i: (0, 0)),
          pl.BlockSpec((H, LBLK), lambda i: (0, i)),
          pl.BlockSpec((1, LBLK), lambda i: (0, i)),
          pl.BlockSpec((64, LBLK), lambda i: (0, i)),
      ],
      out_specs=pl.BlockSpec((1, 1), lambda i: (0, 0)),
      out_shape=jax.ShapeDtypeStruct((1, 1), _f32),
      scratch_shapes=[pltpu.SMEM((4,), _f32)],
  )(s_q, W_lin_pad, b_lin_pad, tpo_pad)


def kernel(t_list, true_prob_s, true_prob_o, edge_index, edge_type,
           ent_embeds, w_rel, W_self, W_ih, W_hh, b_ih, b_hh, W_lin, b_lin):
  del t_list, true_prob_s

  src = edge_index[0].reshape(SEQ_LEN, NS, EPT)
  dst = edge_index[1].reshape(SEQ_LEN, NS, EPT)
  rt = edge_type.reshape(SEQ_LEN, NS, EPT)
  pad = ((0, 0), (0, 0), (0, EPT_PAD - EPT))
  srcp = jnp.pad(src, pad).reshape(-1)
  dstp = jnp.pad(dst, pad, constant_values=-1).reshape(-1)
  rtp = jnp.pad(rt, pad).reshape(-1)

  ent_pad = jnp.pad(ent_embeds, ((0, NPAD - IN_DIM), (0, 0)))
  wrel_flat = w_rel.reshape(-1)

  xself = _tc_xself(ent_pad, W_self)
  gs2 = _sc_aggregate(srcp, dstp, rtp, ent_pad, wrel_flat, xself)
  s_q = _tc_gru(gs2, W_ih, W_hh, b_ih, b_hh)

  W_lin_pad = jnp.pad(W_lin, ((0, 0), (0, LPAD - IN_DIM)))
  b_lin_pad = jnp.pad(b_lin.reshape(1, IN_DIM), ((0, 0), (0, LPAD - IN_DIM)),
                      constant_values=-1e9)
  tpo_pad = jnp.pad(true_prob_o, ((0, 0), (0, LPAD - IN_DIM)))
  loss = _tc_loss(s_q, W_lin_pad, b_lin_pad, tpo_pad)
  return loss[0, 0]


# speedup vs baseline: 3.1084x; 3.1084x over previous
import functools

import jax
import jax.numpy as jnp
from jax import lax
from jax.experimental import pallas as pl
from jax.experimental.pallas import tpu as pltpu
from jax.experimental.pallas import tpu_sc as plsc

IN_DIM = 50000
H = 64
NUM_RELS = 256
SEQ_LEN = 10
E = 800000
ES = E // SEQ_LEN

NC = 2
NS = 16
L = 16

EPT = ES // NS
K = 128
EPT_PAD = 5120
NB = EPT_PAD // K

QTR = 9856
NSUB = 3
TPT = QTR // NS
DUMP = QTR
AGG_ROWS = QTR + 8
NCH = 56
NPAD = 54656

_f32 = jnp.float32


def _sc_aggregate(srcp, dstp, rtp, ent_pad, wrel_flat, xself):
  mesh = plsc.VectorSubcoreMesh(core_axis_name="c", subcore_axis_name="s",
                                num_cores=NC, num_subcores=NS)

  @functools.partial(
      pl.kernel,
      out_type=jax.ShapeDtypeStruct((NC, SEQ_LEN, H), _f32),
      mesh=mesh,
      scratch_types=[
          pltpu.VMEM((EPT_PAD,), jnp.int32),
          pltpu.VMEM((EPT_PAD,), jnp.int32),
          pltpu.VMEM((EPT_PAD,), jnp.int32),
          pltpu.VMEM((EPT_PAD + K + L,), jnp.int32),
          pltpu.VMEM((EPT_PAD + K + L,), jnp.int32),
          pltpu.VMEM((EPT_PAD + K + L,), jnp.int32),
          pltpu.VMEM((NB, K), jnp.int32),
          pltpu.VMEM((NUM_RELS * H,), _f32),
          pltpu.VMEM((2 * K, H), _f32),
          pltpu.VMEM((K, L), _f32),
          pltpu.VMEM((NCH, H), _f32),
          pltpu.VMEM((NCH, L), _f32),
          pltpu.VMEM((NCH, H), _f32),
          pltpu.VMEM((NS, H), _f32),
          pltpu.VMEM((1, H), _f32),
          pltpu.VMEM_SHARED((AGG_ROWS, H), _f32),
          pltpu.VMEM_SHARED((AGG_ROWS, L), _f32),
          pltpu.VMEM_SHARED((NS, H), _f32),
          pltpu.SemaphoreType.DMA,
          pltpu.SemaphoreType.DMA,
      ],
      compiler_params=pltpu.CompilerParams(use_tc_tiling_on_sc=False,
                                           needs_layout_passes=False),
  )
  def body(src_h, dst_h, rt_h, ent_h, wrel_h, xself_h, out_h,
           src_v, rt_v, dstraw_v, csrc_v, crt_v, cdst_v, dst2_v,
           wrel_v, rows_v, ones_v,
           agg_v, deg_v, xs_v, stmax_v, mymax_v,
           agg_sh, deg_sh, stage_sh, sem0, sem1):
    c = lax.axis_index("c")
    t = lax.axis_index("s")

    pltpu.sync_copy(wrel_h, wrel_v)
    one16 = jnp.full((L,), 1.0, dtype=_f32)
    z16 = jnp.zeros((L,), dtype=_f32)

    def fill_ones(n, _):
      ones_v[n, pl.ds(0, L)] = one16
      return 0
    lax.fori_loop(0, K, fill_ones, 0)


    def snapshot(s, _):
      off = (s * NS + t) * EPT_PAD
      pltpu.sync_copy(src_h.at[pl.ds(off, EPT_PAD)], src_v)
      pltpu.sync_copy(rt_h.at[pl.ds(off, EPT_PAD)], rt_v)
      pltpu.sync_copy(dst_h.at[pl.ds(off, EPT_PAD)], dstraw_v)

      def subrange(sub, mxall):
        lo = c * 25000 + sub * QTR
        hi = jnp.minimum(lo + QTR, IN_DIM)

        z16 = jnp.zeros((L,), dtype=_f32)

        def zfill(n, _):
          for k in range(H // L):
            rows_v[n, pl.ds(k * L, L)] = z16
          deg_v[n, pl.ds(0, L)] = z16
          return 0
        lax.fori_loop(0, NCH, zfill, 0)

        def zchunk(i, _):
          r = t * TPT + i * NCH
          pltpu.sync_copy(rows_v.at[pl.ds(0, NCH)], agg_sh.at[pl.ds(r, NCH)])
          pltpu.sync_copy(deg_v, deg_sh.at[pl.ds(r, NCH)])
          return 0
        lax.fori_loop(0, TPT // NCH, zchunk, 0)

        dumps = jnp.full((L,), DUMP, dtype=jnp.int32)
        zi = jnp.zeros((L,), dtype=jnp.int32)

        def cgroup(j, cnt):
          d = dstraw_v[pl.ds(j * L, L)]
          inr = (d >= lo) & (d < hi)
          key = jnp.where(inr, 0, 1)
          ld = jnp.where(inr, d - lo, DUMP)
          _, s_src = plsc.sort_key_val(key, src_v[pl.ds(j * L, L)])
          _, s_rt = plsc.sort_key_val(key, rt_v[pl.ds(j * L, L)])
          _, s_dst = plsc.sort_key_val(key, ld)
          csrc_v[pl.ds(cnt, L)] = s_src
          crt_v[pl.ds(cnt, L)] = s_rt
          cdst_v[pl.ds(cnt, L)] = s_dst
          return cnt + plsc.all_reduce_population_count(inr)[0]
        cnt = lax.fori_loop(0, EPT_PAD // L, cgroup, 0)

        def tfill(j, _):
          csrc_v[pl.ds(cnt + j * L, L)] = zi
          crt_v[pl.ds(cnt + j * L, L)] = zi
          cdst_v[pl.ds(cnt + j * L, L)] = dumps
          return 0
        lax.fori_loop(0, K // L, tfill, 0)
        plsc.subcore_barrier()

        nblk = (cnt + K - 1) // K

        def issue(b):
          @pl.when((b < nblk) & (b % 2 == 0))
          def _():
            pltpu.async_copy(ent_h.at[csrc_v.at[pl.ds(b * K, K)]],
                             rows_v.at[pl.ds(0, K)], sem0)

          @pl.when((b < nblk) & (b % 2 == 1))
          def _():
            pltpu.async_copy(ent_h.at[csrc_v.at[pl.ds(b * K, K)]],
                             rows_v.at[pl.ds(K, K)], sem1)

        issue(0)

        def eblock(b, _):
          @pl.when(b < nblk)
          def _():
            def cpy(jj, _):
              dst2_v[b, pl.ds(jj * L, L)] = cdst_v[pl.ds(b * K + jj * L, L)]
              return 0
            lax.fori_loop(0, K // L, cpy, 0)

            @pl.when(b % 2 == 0)
            def _():
              pltpu.make_async_copy(ent_h.at[csrc_v.at[pl.ds(b * K, K)]],
                                    rows_v.at[pl.ds(0, K)], sem0).wait()

            @pl.when(b % 2 == 1)
            def _():
              pltpu.make_async_copy(ent_h.at[csrc_v.at[pl.ds(b * K, K)]],
                                    rows_v.at[pl.ds(K, K)], sem1).wait()

            issue(b + 1)
            base = (b % 2) * K

            def emul(j, _):
              rt16 = crt_v[pl.ds(b * K + j * L, L)]
              for ee in range(L):
                e = j * L + ee
                wb = rt16[ee] * H
                for k in range(H // L):
                  rows_v[base + e, pl.ds(k * L, L)] = (
                      rows_v[base + e, pl.ds(k * L, L)]
                      * wrel_v[pl.ds(wb + k * L, L)])
              return 0
            lax.fori_loop(0, K // L, emul, 0)

            pltpu.sync_copy(rows_v.at[pl.ds(base, K)],
                            agg_sh.at[dst2_v.at[b]], add=True)
            pltpu.sync_copy(ones_v, deg_sh.at[dst2_v.at[b]], add=True)
          return 0
        lax.fori_loop(0, NB, eblock, 0)
        plsc.subcore_barrier()

        def nchunk(i, mx):
          r = t * TPT + i * NCH
          pltpu.sync_copy(agg_sh.at[pl.ds(r, NCH)], agg_v)
          pltpu.sync_copy(deg_sh.at[pl.ds(r, NCH)], deg_v)
          pltpu.sync_copy(xself_h.at[pl.ds(lo + r, NCH)], xs_v)

          def node(n, mx):
            d16 = deg_v[n, pl.ds(0, L)]
            bc = 1.0 / jnp.maximum(d16, 1.0)
            new = []
            for k in range(H // L):
              v = agg_v[n, pl.ds(k * L, L)] * bc + xs_v[n, pl.ds(k * L, L)]
              v = jnp.maximum(v, 0.0)
              new.append(jnp.maximum(mx[k], v))
            return tuple(new)
          return lax.fori_loop(0, NCH, node, mx)

        mx = lax.fori_loop(0, TPT // NCH, nchunk, mxall)
        plsc.subcore_barrier()
        return mx

      mx0 = tuple(jnp.zeros((L,), dtype=_f32) for _ in range(H // L))
      mx = lax.fori_loop(0, NSUB, subrange, mx0)
      for k in range(H // L):
        mymax_v[0, pl.ds(k * L, L)] = mx[k]
      pltpu.sync_copy(mymax_v, stage_sh.at[pl.ds(t, 1)])
      plsc.subcore_barrier()

      @pl.when(t == 0)
      def _():
        pltpu.sync_copy(stage_sh, stmax_v)

        def tred(u, mr):
          return tuple(
              jnp.maximum(mr[k], stmax_v[u, pl.ds(k * L, L)])
              for k in range(H // L))
        mr = lax.fori_loop(0, NS, tred,
                           tuple(jnp.zeros((L,), dtype=_f32)
                                 for _ in range(H // L)))
        for k in range(H // L):
          mymax_v[0, pl.ds(k * L, L)] = mr[k]
        pltpu.sync_copy(mymax_v, out_h.at[c, pl.ds(s, 1)])
      plsc.subcore_barrier()
      return 0

    lax.fori_loop(0, SEQ_LEN, snapshot, 0)

  return body(srcp, dstp, rtp, ent_pad, wrel_flat, xself)


def _tc_xself(ent_pad, W_self):
  blk = NPAD // 8

  def body(e_ref, w_ref, o_ref):
    o_ref[...] = jnp.dot(e_ref[...], w_ref[...],
                         preferred_element_type=_f32)

  return pl.pallas_call(
      body,
      grid=(8,),
      in_specs=[
          pl.BlockSpec((blk, H), lambda i: (i, 0)),
          pl.BlockSpec((H, H), lambda i: (0, 0)),
      ],
      out_specs=pl.BlockSpec((blk, H), lambda i: (i, 0)),
      out_shape=jax.ShapeDtypeStruct((NPAD, H), _f32),
  )(ent_pad, W_self)


def _tc_gru(gs2, W_ih, W_hh, b_ih, b_hh):
  def body(g_ref, wi_ref, wh_ref, bi_ref, bh_ref, o_ref):
    gs = jnp.maximum(g_ref[0], g_ref[1])
    bi = bi_ref[...]
    bh = bh_ref[...]

    h = jnp.zeros((1, H), dtype=_f32)
    for s in range(SEQ_LEN):
      x = gs[s:s + 1]
      gi = jnp.dot(x, wi_ref[...], preferred_element_type=_f32) + bi
      gh = jnp.dot(h, wh_ref[...], preferred_element_type=_f32) + bh
      ir, iz, in_ = gi[:, :H], gi[:, H:2 * H], gi[:, 2 * H:]
      hr, hz, hn = gh[:, :H], gh[:, H:2 * H], gh[:, 2 * H:]
      r = jax.nn.sigmoid(ir + hr)
      z = jax.nn.sigmoid(iz + hz)
      n = jnp.tanh(in_ + r * hn)
      h = (1.0 - z) * n + z * h
    o_ref[...] = h

  return pl.pallas_call(
      body,
      out_shape=jax.ShapeDtypeStruct((1, H), _f32),
  )(gs2, W_ih, W_hh, b_ih.reshape(1, 3 * H), b_hh.reshape(1, 3 * H))


LPAD = 50176
LBLK = 1024


def _tc_loss(s_q, W_lin_pad, b_lin_pad, tpo_pad):
  nt = LPAD // LBLK

  def body(sq_ref, w_ref, b_ref, t_ref, o_ref, acc):
    i = pl.program_id(0)

    @pl.when(i == 0)
    def _():
      acc[0] = -1e30
      acc[1] = 0.0
      acc[2] = 0.0

    p = (jnp.dot(sq_ref[...], w_ref[...], preferred_element_type=_f32)
         + b_ref[...])
    tb = jnp.sum(t_ref[...], axis=0, keepdims=True) * (1.0 / 64.0)
    m_old = acc[0]
    m_new = jnp.maximum(m_old, jnp.max(p))
    se = acc[1] * jnp.exp(m_old - m_new) + jnp.sum(jnp.exp(p - m_new))
    acc[0] = m_new
    acc[1] = se
    acc[2] = acc[2] + jnp.sum(tb * p)

    @pl.when(i == nt - 1)
    def _():
      o_ref[...] = jnp.full((1, 1), (acc[0] + jnp.log(acc[1])) - acc[2],
                            dtype=_f32)

  return pl.pallas_call(
      body,
      grid=(nt,),
      in_specs=[
          pl.BlockSpec((1, H), lambda i: (0, 0)),
          pl.BlockSpec((H, LBLK), lambda i: (0, i)),
          pl.BlockSpec((1, LBLK), lambda i: (0, i)),
          pl.BlockSpec((64, LBLK), lambda i: (0, i)),
      ],
      out_specs=pl.BlockSpec((1, 1), lambda i: (0, 0)),
      out_shape=jax.ShapeDtypeStruct((1, 1), _f32),
      scratch_shapes=[pltpu.SMEM((4,), _f32)],
  )(s_q, W_lin_pad, b_lin_pad, tpo_pad)


def kernel(t_list, true_prob_s, true_prob_o, edge_index, edge_type,
           ent_embeds, w_rel, W_self, W_ih, W_hh, b_ih, b_hh, W_lin, b_lin):
  del t_list, true_prob_s

  src = edge_index[0].reshape(SEQ_LEN, NS, EPT)
  dst = edge_index[1].reshape(SEQ_LEN, NS, EPT)
  rt = edge_type.reshape(SEQ_LEN, NS, EPT)
  pad = ((0, 0), (0, 0), (0, EPT_PAD - EPT))
  srcp = jnp.pad(src, pad).reshape(-1)
  dstp = jnp.pad(dst, pad, constant_values=-1).reshape(-1)
  rtp = jnp.pad(rt, pad).reshape(-1)

  ent_pad = jnp.pad(ent_embeds, ((0, NPAD - IN_DIM), (0, 0)))
  wrel_flat = w_rel.reshape(-1)

  xself = _tc_xself(ent_pad, W_self)
  gs2 = _sc_aggregate(srcp, dstp, rtp, ent_pad, wrel_flat, xself)
  s_q = _tc_gru(gs2, W_ih, W_hh, b_ih, b_hh)

  W_lin_pad = jnp.pad(W_lin, ((0, 0), (0, LPAD - IN_DIM)))
  b_lin_pad = jnp.pad(b_lin.reshape(1, IN_DIM), ((0, 0), (0, LPAD - IN_DIM)),
                      constant_values=-1e9)
  tpo_pad = jnp.pad(true_prob_o, ((0, 0), (0, LPAD - IN_DIM)))
  loss = _tc_loss(s_q, W_lin_pad, b_lin_pad, tpo_pad)
  return loss[0, 0]


# speedup vs baseline: 4.4576x; 1.4341x over previous
import functools

import jax
import jax.numpy as jnp
from jax import lax
from jax.experimental import pallas as pl
from jax.experimental.pallas import tpu as pltpu
from jax.experimental.pallas import tpu_sc as plsc

IN_DIM = 50000
H = 64
NUM_RELS = 256
SEQ_LEN = 10
E = 800000
ES = E // SEQ_LEN

NC = 2
NS = 16
L = 16

EPT = ES // NS
K = 64
EPT_PAD = 5120
NB = EPT_PAD // K

QTR = 8960
NSUB = 3
TPT = QTR // NS
DUMP = QTR
AGG_ROWS = QTR + 8
NCH = 56
NPAD = 51968
W = H + 8

_f32 = jnp.float32


def _sc_aggregate(srcp, dstp, rtp, ent_pad, wrel_flat, xself):
  mesh = plsc.VectorSubcoreMesh(core_axis_name="c", subcore_axis_name="s",
                                num_cores=NC, num_subcores=NS)

  @functools.partial(
      pl.kernel,
      out_type=jax.ShapeDtypeStruct((NC, SEQ_LEN, H), _f32),
      mesh=mesh,
      scratch_types=[
          pltpu.VMEM((EPT_PAD,), jnp.int32),
          pltpu.VMEM((EPT_PAD,), jnp.int32),
          pltpu.VMEM((EPT_PAD,), jnp.int32),
          pltpu.VMEM((EPT_PAD + K + L,), jnp.int32),
          pltpu.VMEM((EPT_PAD + K + L,), jnp.int32),
          pltpu.VMEM((EPT_PAD + K + L,), jnp.int32),
          pltpu.VMEM((NB, K), jnp.int32),
          pltpu.VMEM((NUM_RELS * H,), _f32),
          pltpu.VMEM((2 * K, H), _f32),
          pltpu.VMEM((2 * K, W), _f32),
          pltpu.VMEM((2 * NCH, W), _f32),
          pltpu.VMEM((2 * NCH, H), _f32),
          pltpu.VMEM((NS, H), _f32),
          pltpu.VMEM((1, H), _f32),
          pltpu.VMEM_SHARED((AGG_ROWS, W), _f32),
          pltpu.VMEM_SHARED((NS, H), _f32),
          pltpu.SemaphoreType.DMA,
          pltpu.SemaphoreType.DMA,
          pltpu.SemaphoreType.DMA,
          pltpu.SemaphoreType.DMA,
      ],
      compiler_params=pltpu.CompilerParams(use_tc_tiling_on_sc=False,
                                           needs_layout_passes=False),
  )
  def body(src_h, dst_h, rt_h, ent_h, wrel_h, xself_h, out_h,
           src_v, rt_v, dstraw_v, csrc_v, crt_v, cdst_v, dst2_v,
           wrel_v, gbuf_v, sbuf_v, agg_v, xs_v, stmax_v, mymax_v,
           agg_sh, stage_sh, sem0, sem1, semS0, semS1):
    c = lax.axis_index("c")
    t = lax.axis_index("s")

    pltpu.sync_copy(wrel_h, wrel_v)
    z16 = jnp.zeros((L,), dtype=_f32)
    mix16 = jnp.where(lax.iota(jnp.int32, L) < 8, 0.0, 1.0).astype(_f32)

    def fill_ones(n, _):
      sbuf_v[n, pl.ds(W - L, L)] = mix16
      return 0
    lax.fori_loop(0, 2 * K, fill_ones, 0)


    def snapshot(s, _):
      off = (s * NS + t) * EPT_PAD
      pltpu.sync_copy(src_h.at[pl.ds(off, EPT_PAD)], src_v)
      pltpu.sync_copy(rt_h.at[pl.ds(off, EPT_PAD)], rt_v)
      pltpu.sync_copy(dst_h.at[pl.ds(off, EPT_PAD)], dstraw_v)

      def subrange(sub, mxall):
        lo = c * 25000 + sub * QTR
        hi = jnp.minimum(lo + QTR, IN_DIM)

        def zfill(n, _):
          for k in range(H // L):
            sbuf_v[n, pl.ds(k * L, L)] = z16
          sbuf_v[n, pl.ds(W - L, L)] = z16
          return 0
        lax.fori_loop(0, NCH, zfill, 0)

        def zchunk(i, _):
          r = t * TPT + i * NCH
          pltpu.async_copy(sbuf_v.at[pl.ds(0, NCH)], agg_sh.at[pl.ds(r, NCH)],
                           semS0)
          return 0
        lax.fori_loop(0, TPT // NCH, zchunk, 0)

        dumps = jnp.full((L,), DUMP, dtype=jnp.int32)
        zi = jnp.zeros((L,), dtype=jnp.int32)

        def cgroup(j, cnt):
          d = dstraw_v[pl.ds(j * L, L)]
          inr = (d >= lo) & (d < hi)
          key = jnp.where(inr, 0, 1)
          ld = jnp.where(inr, d - lo, DUMP)
          _, s_src = plsc.sort_key_val(key, src_v[pl.ds(j * L, L)])
          _, s_rt = plsc.sort_key_val(key, rt_v[pl.ds(j * L, L)])
          _, s_dst = plsc.sort_key_val(key, ld)
          csrc_v[pl.ds(cnt, L)] = s_src
          crt_v[pl.ds(cnt, L)] = s_rt
          cdst_v[pl.ds(cnt, L)] = s_dst
          return cnt + plsc.all_reduce_population_count(inr)[0]
        cnt = lax.fori_loop(0, EPT_PAD // L, cgroup, 0)

        def zdrain(i, _):
          pltpu.make_async_copy(sbuf_v.at[pl.ds(0, NCH)],
                                agg_sh.at[pl.ds(t * TPT, NCH)], semS0).wait()
          return 0
        lax.fori_loop(0, TPT // NCH, zdrain, 0)

        def refix(n, _):
          sbuf_v[n, pl.ds(W - L, L)] = mix16
          return 0
        lax.fori_loop(0, NCH, refix, 0)

        def tfill(j, _):
          csrc_v[pl.ds(cnt + j * L, L)] = zi
          crt_v[pl.ds(cnt + j * L, L)] = zi
          cdst_v[pl.ds(cnt + j * L, L)] = dumps
          return 0
        lax.fori_loop(0, K // L, tfill, 0)
        plsc.subcore_barrier()

        nblk = (cnt + K - 1) // K

        def issue(b):
          @pl.when((b < nblk) & (b % 2 == 0))
          def _():
            pltpu.async_copy(ent_h.at[csrc_v.at[pl.ds(b * K, K)]],
                             gbuf_v.at[pl.ds(0, K)], sem0)

          @pl.when((b < nblk) & (b % 2 == 1))
          def _():
            pltpu.async_copy(ent_h.at[csrc_v.at[pl.ds(b * K, K)]],
                             gbuf_v.at[pl.ds(K, K)], sem1)

        def wait_scat(b):
          @pl.when((b >= 0) & (b % 2 == 0))
          def _():
            pltpu.make_async_copy(sbuf_v.at[pl.ds(0, K)],
                                  agg_sh.at[dst2_v.at[0]], semS0).wait()

          @pl.when((b >= 0) & (b % 2 == 1))
          def _():
            pltpu.make_async_copy(sbuf_v.at[pl.ds(K, K)],
                                  agg_sh.at[dst2_v.at[0]], semS1).wait()

        issue(0)

        def eblock(b, _):
          @pl.when(b < nblk)
          def _():
            def cpy(jj, _):
              dst2_v[b, pl.ds(jj * L, L)] = cdst_v[pl.ds(b * K + jj * L, L)]
              return 0
            lax.fori_loop(0, K // L, cpy, 0)

            @pl.when(b % 2 == 0)
            def _():
              pltpu.make_async_copy(ent_h.at[csrc_v.at[pl.ds(b * K, K)]],
                                    gbuf_v.at[pl.ds(0, K)], sem0).wait()

            @pl.when(b % 2 == 1)
            def _():
              pltpu.make_async_copy(ent_h.at[csrc_v.at[pl.ds(b * K, K)]],
                                    gbuf_v.at[pl.ds(K, K)], sem1).wait()

            issue(b + 1)
            wait_scat(b - 2)
            base = (b % 2) * K

            def emul(j, _):
              rt16 = crt_v[pl.ds(b * K + j * L, L)]
              for ee in range(L):
                e = j * L + ee
                wb = rt16[ee] * H
                for k in range(H // L):
                  sbuf_v[base + e, pl.ds(k * L, L)] = (
                      gbuf_v[base + e, pl.ds(k * L, L)]
                      * wrel_v[pl.ds(wb + k * L, L)])
              return 0
            lax.fori_loop(0, K // L, emul, 0)

            @pl.when(b % 2 == 0)
            def _():
              pltpu.async_copy(sbuf_v.at[pl.ds(0, K)],
                               agg_sh.at[dst2_v.at[b]], semS0, add=True)

            @pl.when(b % 2 == 1)
            def _():
              pltpu.async_copy(sbuf_v.at[pl.ds(K, K)],
                               agg_sh.at[dst2_v.at[b]], semS1, add=True)
          return 0
        lax.fori_loop(0, NB, eblock, 0)
        wait_scat(nblk - 2)
        wait_scat(nblk - 1)
        plsc.subcore_barrier()

        nchunks = TPT // NCH

        def issue_n(i):
          r = t * TPT + i * NCH

          @pl.when((i < nchunks) & (i % 2 == 0))
          def _():
            pltpu.async_copy(agg_sh.at[pl.ds(r, NCH)],
                             agg_v.at[pl.ds(0, NCH)], sem0)
            pltpu.async_copy(xself_h.at[pl.ds(lo + r, NCH)],
                             xs_v.at[pl.ds(0, NCH)], semS0)

          @pl.when((i < nchunks) & (i % 2 == 1))
          def _():
            pltpu.async_copy(agg_sh.at[pl.ds(r, NCH)],
                             agg_v.at[pl.ds(NCH, NCH)], sem1)
            pltpu.async_copy(xself_h.at[pl.ds(lo + r, NCH)],
                             xs_v.at[pl.ds(NCH, NCH)], semS1)

        issue_n(0)

        def nchunk(i, mx):
          r = t * TPT + i * NCH

          @pl.when(i % 2 == 0)
          def _():
            pltpu.make_async_copy(agg_sh.at[pl.ds(r, NCH)],
                                  agg_v.at[pl.ds(0, NCH)], sem0).wait()
            pltpu.make_async_copy(xself_h.at[pl.ds(lo + r, NCH)],
                                  xs_v.at[pl.ds(0, NCH)], semS0).wait()

          @pl.when(i % 2 == 1)
          def _():
            pltpu.make_async_copy(agg_sh.at[pl.ds(r, NCH)],
                                  agg_v.at[pl.ds(NCH, NCH)], sem1).wait()
            pltpu.make_async_copy(xself_h.at[pl.ds(lo + r, NCH)],
                                  xs_v.at[pl.ds(NCH, NCH)], semS1).wait()

          issue_n(i + 1)
          nb = (i % 2) * NCH

          def node(n, mx):
            dtail = agg_v[nb + n, pl.ds(W - L, L)]
            dinv = (1.0 / jnp.maximum(dtail, 1.0))[8]
            new = []
            for k in range(H // L):
              v = (agg_v[nb + n, pl.ds(k * L, L)] * dinv
                   + xs_v[nb + n, pl.ds(k * L, L)])
              v = jnp.maximum(v, 0.0)
              new.append(jnp.maximum(mx[k], v))
            return tuple(new)
          return lax.fori_loop(0, NCH, node, mx)

        mx = lax.fori_loop(0, nchunks, nchunk, mxall)
        plsc.subcore_barrier()
        return mx

      mx0 = tuple(jnp.zeros((L,), dtype=_f32) for _ in range(H // L))
      mx = lax.fori_loop(0, NSUB, subrange, mx0)
      for k in range(H // L):
        mymax_v[0, pl.ds(k * L, L)] = mx[k]
      pltpu.sync_copy(mymax_v, stage_sh.at[pl.ds(t, 1)])
      plsc.subcore_barrier()

      @pl.when(t == 0)
      def _():
        pltpu.sync_copy(stage_sh, stmax_v)

        def tred(u, mr):
          return tuple(
              jnp.maximum(mr[k], stmax_v[u, pl.ds(k * L, L)])
              for k in range(H // L))
        mr = lax.fori_loop(0, NS, tred,
                           tuple(jnp.zeros((L,), dtype=_f32)
                                 for _ in range(H // L)))
        for k in range(H // L):
          mymax_v[0, pl.ds(k * L, L)] = mr[k]
        pltpu.sync_copy(mymax_v, out_h.at[c, pl.ds(s, 1)])
      plsc.subcore_barrier()
      return 0

    lax.fori_loop(0, SEQ_LEN, snapshot, 0)

  return body(srcp, dstp, rtp, ent_pad, wrel_flat, xself)


def _tc_xself(ent_pad, W_self):
  blk = NPAD // 8

  def body(e_ref, w_ref, o_ref):
    o_ref[...] = jnp.dot(e_ref[...], w_ref[...],
                         preferred_element_type=_f32)

  return pl.pallas_call(
      body,
      grid=(8,),
      in_specs=[
          pl.BlockSpec((blk, H), lambda i: (i, 0)),
          pl.BlockSpec((H, H), lambda i: (0, 0)),
      ],
      out_specs=pl.BlockSpec((blk, H), lambda i: (i, 0)),
      out_shape=jax.ShapeDtypeStruct((NPAD, H), _f32),
  )(ent_pad, W_self)


def _tc_gru(gs2, W_ih, W_hh, b_ih, b_hh):
  def body(g_ref, wi_ref, wh_ref, bi_ref, bh_ref, o_ref):
    gs = jnp.maximum(g_ref[0], g_ref[1])
    bi = bi_ref[...]
    bh = bh_ref[...]

    h = jnp.zeros((1, H), dtype=_f32)
    for s in range(SEQ_LEN):
      x = gs[s:s + 1]
      gi = jnp.dot(x, wi_ref[...], preferred_element_type=_f32) + bi
      gh = jnp.dot(h, wh_ref[...], preferred_element_type=_f32) + bh
      ir, iz, in_ = gi[:, :H], gi[:, H:2 * H], gi[:, 2 * H:]
      hr, hz, hn = gh[:, :H], gh[:, H:2 * H], gh[:, 2 * H:]
      r = jax.nn.sigmoid(ir + hr)
      z = jax.nn.sigmoid(iz + hz)
      n = jnp.tanh(in_ + r * hn)
      h = (1.0 - z) * n + z * h
    o_ref[...] = h

  return pl.pallas_call(
      body,
      out_shape=jax.ShapeDtypeStruct((1, H), _f32),
  )(gs2, W_ih, W_hh, b_ih.reshape(1, 3 * H), b_hh.reshape(1, 3 * H))


LPAD = 50176
LBLK = 1024


def _tc_loss(s_q, W_lin_pad, b_lin_pad, tpo_pad):
  nt = LPAD // LBLK

  def body(sq_ref, w_ref, b_ref, t_ref, o_ref, acc):
    i = pl.program_id(0)

    @pl.when(i == 0)
    def _():
      acc[0] = -1e30
      acc[1] = 0.0
      acc[2] = 0.0

    p = (jnp.dot(sq_ref[...], w_ref[...], preferred_element_type=_f32)
         + b_ref[...])
    tb = jnp.sum(t_ref[...], axis=0, keepdims=True) * (1.0 / 64.0)
    m_old = acc[0]
    m_new = jnp.maximum(m_old, jnp.max(p))
    se = acc[1] * jnp.exp(m_old - m_new) + jnp.sum(jnp.exp(p - m_new))
    acc[0] = m_new
    acc[1] = se
    acc[2] = acc[2] + jnp.sum(tb * p)

    @pl.when(i == nt - 1)
    def _():
      o_ref[...] = jnp.full((1, 1), (acc[0] + jnp.log(acc[1])) - acc[2],
                            dtype=_f32)

  return pl.pallas_call(
      body,
      grid=(nt,),
      in_specs=[
          pl.BlockSpec((1, H), lambda i: (0, 0)),
          pl.BlockSpec((H, LBLK), lambda i: (0, i)),
          pl.BlockSpec((1, LBLK), lambda i: (0, i)),
          pl.BlockSpec((64, LBLK), lambda i: (0, i)),
      ],
      out_specs=pl.BlockSpec((1, 1), lambda i: (0, 0)),
      out_shape=jax.ShapeDtypeStruct((1, 1), _f32),
      scratch_shapes=[pltpu.SMEM((4,), _f32)],
  )(s_q, W_lin_pad, b_lin_pad, tpo_pad)


def kernel(t_list, true_prob_s, true_prob_o, edge_index, edge_type,
           ent_embeds, w_rel, W_self, W_ih, W_hh, b_ih, b_hh, W_lin, b_lin):
  del t_list, true_prob_s

  src = edge_index[0].reshape(SEQ_LEN, NS, EPT)
  dst = edge_index[1].reshape(SEQ_LEN, NS, EPT)
  rt = edge_type.reshape(SEQ_LEN, NS, EPT)
  pad = ((0, 0), (0, 0), (0, EPT_PAD - EPT))
  srcp = jnp.pad(src, pad).reshape(-1)
  dstp = jnp.pad(dst, pad, constant_values=-1).reshape(-1)
  rtp = jnp.pad(rt, pad).reshape(-1)

  ent_pad = jnp.pad(ent_embeds, ((0, NPAD - IN_DIM), (0, 0)))
  wrel_flat = w_rel.reshape(-1)

  xself = _tc_xself(ent_pad, W_self)
  gs2 = _sc_aggregate(srcp, dstp, rtp, ent_pad, wrel_flat, xself)
  s_q = _tc_gru(gs2, W_ih, W_hh, b_ih, b_hh)

  W_lin_pad = jnp.pad(W_lin, ((0, 0), (0, LPAD - IN_DIM)))
  b_lin_pad = jnp.pad(b_lin.reshape(1, IN_DIM), ((0, 0), (0, LPAD - IN_DIM)),
                      constant_values=-1e9)
  tpo_pad = jnp.pad(true_prob_o, ((0, 0), (0, LPAD - IN_DIM)))
  loss = _tc_loss(s_q, W_lin_pad, b_lin_pad, tpo_pad)
  return loss[0, 0]


# speedup vs baseline: 4.4666x; 1.0020x over previous
import functools

import jax
import jax.numpy as jnp
from jax import lax
from jax.experimental import pallas as pl
from jax.experimental.pallas import tpu as pltpu
from jax.experimental.pallas import tpu_sc as plsc

IN_DIM = 50000
H = 64
NUM_RELS = 256
SEQ_LEN = 10
E = 800000
ES = E // SEQ_LEN

NC = 2
NS = 16
L = 16

EPT = ES // NS
K = 64
EPT_PAD = 5120
NB = EPT_PAD // K

QTR = 8960
NSUB = 3
TPT = QTR // NS
DUMP = QTR
AGG_ROWS = QTR + 8
NCH = 56
NPAD = 51968
W = H + 8

_f32 = jnp.float32


def _sc_aggregate(srcp, dstp, rtp, ent_pad, wrel_flat, xself):
  mesh = plsc.VectorSubcoreMesh(core_axis_name="c", subcore_axis_name="s",
                                num_cores=NC, num_subcores=NS)

  @functools.partial(
      pl.kernel,
      out_type=jax.ShapeDtypeStruct((NC, NS, SEQ_LEN, H), _f32),
      mesh=mesh,
      scratch_types=[
          pltpu.VMEM((EPT_PAD,), jnp.int32),
          pltpu.VMEM((EPT_PAD,), jnp.int32),
          pltpu.VMEM((EPT_PAD,), jnp.int32),
          pltpu.VMEM((EPT_PAD + K + L,), jnp.int32),
          pltpu.VMEM((EPT_PAD + K + L,), jnp.int32),
          pltpu.VMEM((EPT_PAD + K + L,), jnp.int32),
          pltpu.VMEM((NB, K), jnp.int32),
          pltpu.VMEM((NUM_RELS * H,), _f32),
          pltpu.VMEM((2 * K, H), _f32),
          pltpu.VMEM((2 * K, W), _f32),
          pltpu.VMEM((2 * NCH, W), _f32),
          pltpu.VMEM((2 * NCH, H), _f32),
          pltpu.VMEM((1, H), _f32),
          pltpu.VMEM_SHARED((AGG_ROWS, W), _f32),
          pltpu.SemaphoreType.DMA,
          pltpu.SemaphoreType.DMA,
          pltpu.SemaphoreType.DMA,
          pltpu.SemaphoreType.DMA,
      ],
      compiler_params=pltpu.CompilerParams(use_tc_tiling_on_sc=False,
                                           needs_layout_passes=False),
  )
  def body(src_h, dst_h, rt_h, ent_h, wrel_h, xself_h, out_h,
           src_v, rt_v, dstraw_v, csrc_v, crt_v, cdst_v, dst2_v,
           wrel_v, gbuf_v, sbuf_v, agg_v, xs_v, mymax_v,
           agg_sh, sem0, sem1, semS0, semS1):
    c = lax.axis_index("c")
    t = lax.axis_index("s")

    pltpu.sync_copy(wrel_h, wrel_v)
    z16 = jnp.zeros((L,), dtype=_f32)
    mix16 = jnp.where(lax.iota(jnp.int32, L) < 8, 0.0, 1.0).astype(_f32)

    def fill_ones(n, _):
      sbuf_v[n, pl.ds(W - L, L)] = mix16
      return 0
    lax.fori_loop(0, 2 * K, fill_ones, 0)


    def snapshot(s, _):
      off = (s * NS + t) * EPT_PAD
      pltpu.sync_copy(src_h.at[pl.ds(off, EPT_PAD)], src_v)
      pltpu.sync_copy(rt_h.at[pl.ds(off, EPT_PAD)], rt_v)
      pltpu.sync_copy(dst_h.at[pl.ds(off, EPT_PAD)], dstraw_v)

      def subrange(sub, mxall):
        lo = c * 25000 + sub * QTR
        hi = jnp.minimum(lo + QTR, IN_DIM)

        def zfill(n, _):
          for k in range(H // L):
            sbuf_v[n, pl.ds(k * L, L)] = z16
          sbuf_v[n, pl.ds(W - L, L)] = z16
          return 0
        lax.fori_loop(0, NCH, zfill, 0)

        def zchunk(i, _):
          r = t * TPT + i * NCH
          pltpu.async_copy(sbuf_v.at[pl.ds(0, NCH)], agg_sh.at[pl.ds(r, NCH)],
                           semS0)
          return 0
        lax.fori_loop(0, TPT // NCH, zchunk, 0)

        dumps = jnp.full((L,), DUMP, dtype=jnp.int32)
        zi = jnp.zeros((L,), dtype=jnp.int32)

        def cgroup(j, cnt):
          d = dstraw_v[pl.ds(j * L, L)]
          inr = (d >= lo) & (d < hi)
          key = jnp.where(inr, 0, 1)
          ld = jnp.where(inr, d - lo, DUMP)
          _, s_src = plsc.sort_key_val(key, src_v[pl.ds(j * L, L)])
          _, s_rt = plsc.sort_key_val(key, rt_v[pl.ds(j * L, L)])
          _, s_dst = plsc.sort_key_val(key, ld)
          csrc_v[pl.ds(cnt, L)] = s_src
          crt_v[pl.ds(cnt, L)] = s_rt
          cdst_v[pl.ds(cnt, L)] = s_dst
          return cnt + plsc.all_reduce_population_count(inr)[0]
        cnt = lax.fori_loop(0, EPT_PAD // L, cgroup, 0)

        def zdrain(i, _):
          pltpu.make_async_copy(sbuf_v.at[pl.ds(0, NCH)],
                                agg_sh.at[pl.ds(t * TPT, NCH)], semS0).wait()
          return 0
        lax.fori_loop(0, TPT // NCH, zdrain, 0)

        def refix(n, _):
          sbuf_v[n, pl.ds(W - L, L)] = mix16
          return 0
        lax.fori_loop(0, NCH, refix, 0)

        def tfill(j, _):
          csrc_v[pl.ds(cnt + j * L, L)] = zi
          crt_v[pl.ds(cnt + j * L, L)] = zi
          cdst_v[pl.ds(cnt + j * L, L)] = dumps
          return 0
        lax.fori_loop(0, K // L, tfill, 0)
        plsc.subcore_barrier()

        nblk = (cnt + K - 1) // K

        def issue(b):
          @pl.when((b < nblk) & (b % 2 == 0))
          def _():
            pltpu.async_copy(ent_h.at[csrc_v.at[pl.ds(b * K, K)]],
                             gbuf_v.at[pl.ds(0, K)], sem0)

          @pl.when((b < nblk) & (b % 2 == 1))
          def _():
            pltpu.async_copy(ent_h.at[csrc_v.at[pl.ds(b * K, K)]],
                             gbuf_v.at[pl.ds(K, K)], sem1)

        def wait_scat(b):
          @pl.when((b >= 0) & (b % 2 == 0))
          def _():
            pltpu.make_async_copy(sbuf_v.at[pl.ds(0, K)],
                                  agg_sh.at[dst2_v.at[0]], semS0).wait()

          @pl.when((b >= 0) & (b % 2 == 1))
          def _():
            pltpu.make_async_copy(sbuf_v.at[pl.ds(K, K)],
                                  agg_sh.at[dst2_v.at[0]], semS1).wait()

        issue(0)

        def eblock(b, _):
          @pl.when(b < nblk)
          def _():
            def cpy(jj, _):
              dst2_v[b, pl.ds(jj * L, L)] = cdst_v[pl.ds(b * K + jj * L, L)]
              return 0
            lax.fori_loop(0, K // L, cpy, 0)

            @pl.when(b % 2 == 0)
            def _():
              pltpu.make_async_copy(ent_h.at[csrc_v.at[pl.ds(b * K, K)]],
                                    gbuf_v.at[pl.ds(0, K)], sem0).wait()

            @pl.when(b % 2 == 1)
            def _():
              pltpu.make_async_copy(ent_h.at[csrc_v.at[pl.ds(b * K, K)]],
                                    gbuf_v.at[pl.ds(K, K)], sem1).wait()

            issue(b + 1)
            wait_scat(b - 2)
            base = (b % 2) * K

            def emul(j, _):
              rt16 = crt_v[pl.ds(b * K + j * L, L)]
              for ee in range(L):
                e = j * L + ee
                wb = rt16[ee] * H
                for k in range(H // L):
                  sbuf_v[base + e, pl.ds(k * L, L)] = (
                      gbuf_v[base + e, pl.ds(k * L, L)]
                      * wrel_v[pl.ds(wb + k * L, L)])
              return 0
            lax.fori_loop(0, K // L, emul, 0)

            @pl.when(b % 2 == 0)
            def _():
              pltpu.async_copy(sbuf_v.at[pl.ds(0, K)],
                               agg_sh.at[dst2_v.at[b]], semS0, add=True)

            @pl.when(b % 2 == 1)
            def _():
              pltpu.async_copy(sbuf_v.at[pl.ds(K, K)],
                               agg_sh.at[dst2_v.at[b]], semS1, add=True)
          return 0
        lax.fori_loop(0, NB, eblock, 0)
        wait_scat(nblk - 2)
        wait_scat(nblk - 1)
        plsc.subcore_barrier()

        nchunks = TPT // NCH

        def issue_n(i):
          r = t * TPT + i * NCH

          @pl.when((i < nchunks) & (i % 2 == 0))
          def _():
            pltpu.async_copy(agg_sh.at[pl.ds(r, NCH)],
                             agg_v.at[pl.ds(0, NCH)], sem0)
            pltpu.async_copy(xself_h.at[pl.ds(lo + r, NCH)],
                             xs_v.at[pl.ds(0, NCH)], semS0)

          @pl.when((i < nchunks) & (i % 2 == 1))
          def _():
            pltpu.async_copy(agg_sh.at[pl.ds(r, NCH)],
                             agg_v.at[pl.ds(NCH, NCH)], sem1)
            pltpu.async_copy(xself_h.at[pl.ds(lo + r, NCH)],
                             xs_v.at[pl.ds(NCH, NCH)], semS1)

        issue_n(0)

        def nchunk(i, mx):
          r = t * TPT + i * NCH

          @pl.when(i % 2 == 0)
          def _():
            pltpu.make_async_copy(agg_sh.at[pl.ds(r, NCH)],
                                  agg_v.at[pl.ds(0, NCH)], sem0).wait()
            pltpu.make_async_copy(xself_h.at[pl.ds(lo + r, NCH)],
                                  xs_v.at[pl.ds(0, NCH)], semS0).wait()

          @pl.when(i % 2 == 1)
          def _():
            pltpu.make_async_copy(agg_sh.at[pl.ds(r, NCH)],
                                  agg_v.at[pl.ds(NCH, NCH)], sem1).wait()
            pltpu.make_async_copy(xself_h.at[pl.ds(lo + r, NCH)],
                                  xs_v.at[pl.ds(NCH, NCH)], semS1).wait()

          issue_n(i + 1)
          nb = (i % 2) * NCH

          def node(n, mx):
            dtail = agg_v[nb + n, pl.ds(W - L, L)]
            dinv = (1.0 / jnp.maximum(dtail, 1.0))[8]
            new = []
            for k in range(H // L):
              v = (agg_v[nb + n, pl.ds(k * L, L)] * dinv
                   + xs_v[nb + n, pl.ds(k * L, L)])
              v = jnp.maximum(v, 0.0)
              new.append(jnp.maximum(mx[k], v))
            return tuple(new)
          return lax.fori_loop(0, NCH, node, mx)

        mx = lax.fori_loop(0, nchunks, nchunk, mxall)
        return mx

      mx0 = tuple(jnp.zeros((L,), dtype=_f32) for _ in range(H // L))
      mx = lax.fori_loop(0, NSUB, subrange, mx0)
      for k in range(H // L):
        mymax_v[0, pl.ds(k * L, L)] = mx[k]
      pltpu.sync_copy(mymax_v, out_h.at[c, t, pl.ds(s, 1)])
      return 0

    lax.fori_loop(0, SEQ_LEN, snapshot, 0)

  return body(srcp, dstp, rtp, ent_pad, wrel_flat, xself)


def _tc_xself(ent_pad, W_self):
  blk = NPAD // 8

  def body(e_ref, w_ref, o_ref):
    o_ref[...] = jnp.dot(e_ref[...], w_ref[...],
                         preferred_element_type=_f32)

  return pl.pallas_call(
      body,
      grid=(8,),
      in_specs=[
          pl.BlockSpec((blk, H), lambda i: (i, 0)),
          pl.BlockSpec((H, H), lambda i: (0, 0)),
      ],
      out_specs=pl.BlockSpec((blk, H), lambda i: (i, 0)),
      out_shape=jax.ShapeDtypeStruct((NPAD, H), _f32),
  )(ent_pad, W_self)


def _tc_gru(gs2, W_ih, W_hh, b_ih, b_hh):
  def body(g_ref, wi_ref, wh_ref, bi_ref, bh_ref, o_ref):
    gs = jnp.max(g_ref[...], axis=(0, 1))
    bi = bi_ref[...]
    bh = bh_ref[...]

    h = jnp.zeros((1, H), dtype=_f32)
    for s in range(SEQ_LEN):
      x = gs[s:s + 1]
      gi = jnp.dot(x, wi_ref[...], preferred_element_type=_f32) + bi
      gh = jnp.dot(h, wh_ref[...], preferred_element_type=_f32) + bh
      ir, iz, in_ = gi[:, :H], gi[:, H:2 * H], gi[:, 2 * H:]
      hr, hz, hn = gh[:, :H], gh[:, H:2 * H], gh[:, 2 * H:]
      r = jax.nn.sigmoid(ir + hr)
      z = jax.nn.sigmoid(iz + hz)
      n = jnp.tanh(in_ + r * hn)
      h = (1.0 - z) * n + z * h
    o_ref[...] = h

  return pl.pallas_call(
      body,
      out_shape=jax.ShapeDtypeStruct((1, H), _f32),
  )(gs2, W_ih, W_hh, b_ih.reshape(1, 3 * H), b_hh.reshape(1, 3 * H))


LPAD = 50176
LBLK = 1024


def _tc_loss(s_q, W_lin_pad, b_lin_pad, tpo_pad):
  nt = LPAD // LBLK

  def body(sq_ref, w_ref, b_ref, t_ref, o_ref, acc):
    i = pl.program_id(0)

    @pl.when(i == 0)
    def _():
      acc[0] = -1e30
      acc[1] = 0.0
      acc[2] = 0.0

    p = (jnp.dot(sq_ref[...], w_ref[...], preferred_element_type=_f32)
         + b_ref[...])
    tb = jnp.sum(t_ref[...], axis=0, keepdims=True) * (1.0 / 64.0)
    m_old = acc[0]
    m_new = jnp.maximum(m_old, jnp.max(p))
    se = acc[1] * jnp.exp(m_old - m_new) + jnp.sum(jnp.exp(p - m_new))
    acc[0] = m_new
    acc[1] = se
    acc[2] = acc[2] + jnp.sum(tb * p)

    @pl.when(i == nt - 1)
    def _():
      o_ref[...] = jnp.full((1, 1), (acc[0] + jnp.log(acc[1])) - acc[2],
                            dtype=_f32)

  return pl.pallas_call(
      body,
      grid=(nt,),
      in_specs=[
          pl.BlockSpec((1, H), lambda i: (0, 0)),
          pl.BlockSpec((H, LBLK), lambda i: (0, i)),
          pl.BlockSpec((1, LBLK), lambda i: (0, i)),
          pl.BlockSpec((64, LBLK), lambda i: (0, i)),
      ],
      out_specs=pl.BlockSpec((1, 1), lambda i: (0, 0)),
      out_shape=jax.ShapeDtypeStruct((1, 1), _f32),
      scratch_shapes=[pltpu.SMEM((4,), _f32)],
  )(s_q, W_lin_pad, b_lin_pad, tpo_pad)


def kernel(t_list, true_prob_s, true_prob_o, edge_index, edge_type,
           ent_embeds, w_rel, W_self, W_ih, W_hh, b_ih, b_hh, W_lin, b_lin):
  del t_list, true_prob_s

  src = edge_index[0].reshape(SEQ_LEN, NS, EPT)
  dst = edge_index[1].reshape(SEQ_LEN, NS, EPT)
  rt = edge_type.reshape(SEQ_LEN, NS, EPT)
  pad = ((0, 0), (0, 0), (0, EPT_PAD - EPT))
  srcp = jnp.pad(src, pad).reshape(-1)
  dstp = jnp.pad(dst, pad, constant_values=-1).reshape(-1)
  rtp = jnp.pad(rt, pad).reshape(-1)

  ent_pad = jnp.pad(ent_embeds, ((0, NPAD - IN_DIM), (0, 0)))
  wrel_flat = w_rel.reshape(-1)

  xself = _tc_xself(ent_pad, W_self)
  gs2 = _sc_aggregate(srcp, dstp, rtp, ent_pad, wrel_flat, xself)
  s_q = _tc_gru(gs2, W_ih, W_hh, b_ih, b_hh)

  W_lin_pad = jnp.pad(W_lin, ((0, 0), (0, LPAD - IN_DIM)))
  b_lin_pad = jnp.pad(b_lin.reshape(1, IN_DIM), ((0, 0), (0, LPAD - IN_DIM)),
                      constant_values=-1e9)
  tpo_pad = jnp.pad(true_prob_o, ((0, 0), (0, LPAD - IN_DIM)))
  loss = _tc_loss(s_q, W_lin_pad, b_lin_pad, tpo_pad)
  return loss[0, 0]


# speedup vs baseline: 4.4726x; 1.0013x over previous
import functools

import jax
import jax.numpy as jnp
from jax import lax
from jax.experimental import pallas as pl
from jax.experimental.pallas import tpu as pltpu
from jax.experimental.pallas import tpu_sc as plsc

IN_DIM = 50000
H = 64
NUM_RELS = 256
SEQ_LEN = 10
E = 800000
ES = E // SEQ_LEN

NC = 2
NS = 16
L = 16

EPT = ES // NS
K = 64
EPT_PAD = 5120
NB = EPT_PAD // K

QTR = 8960
NSUB = 3
TPT = QTR // NS
DUMP = QTR
AGG_ROWS = QTR + 8
NCH = 56
NPAD = 51968
W = H + 8

_f32 = jnp.float32


def _sc_aggregate(srcp, dstp, rtp, ent_pad, wrel_flat, xself):
  mesh = plsc.VectorSubcoreMesh(core_axis_name="c", subcore_axis_name="s",
                                num_cores=NC, num_subcores=NS)

  @functools.partial(
      pl.kernel,
      out_type=jax.ShapeDtypeStruct((NC, NS, SEQ_LEN, H), _f32),
      mesh=mesh,
      scratch_types=[
          pltpu.VMEM((EPT_PAD,), jnp.int32),
          pltpu.VMEM((EPT_PAD,), jnp.int32),
          pltpu.VMEM((EPT_PAD,), jnp.int32),
          pltpu.VMEM((EPT_PAD + K + L,), jnp.int32),
          pltpu.VMEM((EPT_PAD + K + L,), jnp.int32),
          pltpu.VMEM((EPT_PAD + K + L,), jnp.int32),
          pltpu.VMEM((NB, K), jnp.int32),
          pltpu.VMEM((NUM_RELS * H,), _f32),
          pltpu.VMEM((2 * K, H), _f32),
          pltpu.VMEM((2 * K, W), _f32),
          pltpu.VMEM((2 * NCH, W), _f32),
          pltpu.VMEM((2 * NCH, H), _f32),
          pltpu.VMEM((1, H), _f32),
          pltpu.VMEM_SHARED((AGG_ROWS, W), _f32),
          pltpu.SemaphoreType.DMA,
          pltpu.SemaphoreType.DMA,
          pltpu.SemaphoreType.DMA,
          pltpu.SemaphoreType.DMA,
      ],
      compiler_params=pltpu.CompilerParams(use_tc_tiling_on_sc=False,
                                           needs_layout_passes=False),
  )
  def body(src_h, dst_h, rt_h, ent_h, wrel_h, xself_h, out_h,
           src_v, rt_v, dstraw_v, csrc_v, crt_v, cdst_v, dst2_v,
           wrel_v, gbuf_v, sbuf_v, agg_v, xs_v, mymax_v,
           agg_sh, sem0, sem1, semS0, semS1):
    c = lax.axis_index("c")
    t = lax.axis_index("s")

    pltpu.sync_copy(wrel_h, wrel_v)
    z16 = jnp.zeros((L,), dtype=_f32)
    mix16 = jnp.where(lax.iota(jnp.int32, L) < 8, 0.0, 1.0).astype(_f32)

    def fill_ones(n, _):
      sbuf_v[n, pl.ds(W - L, L)] = mix16
      return 0
    lax.fori_loop(0, 2 * K, fill_ones, 0)


    def snapshot(s, _):
      off = (s * NS + t) * EPT_PAD
      pltpu.sync_copy(src_h.at[pl.ds(off, EPT_PAD)], src_v)
      pltpu.sync_copy(rt_h.at[pl.ds(off, EPT_PAD)], rt_v)
      pltpu.sync_copy(dst_h.at[pl.ds(off, EPT_PAD)], dstraw_v)

      def subrange(sub, mxall):
        lo = c * 25000 + sub * QTR
        hi = jnp.minimum(lo + QTR, IN_DIM)

        def zfill(n, _):
          for k in range(H // L):
            sbuf_v[n, pl.ds(k * L, L)] = z16
          sbuf_v[n, pl.ds(W - L, L)] = z16
          return 0
        lax.fori_loop(0, NCH, zfill, 0)

        def zchunk(i, _):
          r = t * TPT + i * NCH
          pltpu.async_copy(sbuf_v.at[pl.ds(0, NCH)], agg_sh.at[pl.ds(r, NCH)],
                           semS0)
          return 0
        lax.fori_loop(0, TPT // NCH, zchunk, 0)

        dumps = jnp.full((L,), DUMP, dtype=jnp.int32)
        zi = jnp.zeros((L,), dtype=jnp.int32)

        def cgroup(j, cnt):
          d = dstraw_v[pl.ds(j * L, L)]
          inr = (d >= lo) & (d < hi)
          key = jnp.where(inr, 0, 1)
          ld = jnp.where(inr, d - lo, DUMP)
          _, perm = plsc.sort_key_val(key, lax.iota(jnp.int32, L))

          def vperm(x):
            return lax.gather(
                x, perm[:, None],
                lax.GatherDimensionNumbers(offset_dims=(),
                                           collapsed_slice_dims=(0,),
                                           start_index_map=(0,)),
                (1,), mode=lax.GatherScatterMode.PROMISE_IN_BOUNDS)
          s_src = vperm(src_v[pl.ds(j * L, L)])
          s_rt = vperm(rt_v[pl.ds(j * L, L)])
          s_dst = vperm(ld)
          csrc_v[pl.ds(cnt, L)] = s_src
          crt_v[pl.ds(cnt, L)] = s_rt
          cdst_v[pl.ds(cnt, L)] = s_dst
          return cnt + plsc.all_reduce_population_count(inr)[0]
        cnt = lax.fori_loop(0, EPT_PAD // L, cgroup, 0)

        def zdrain(i, _):
          pltpu.make_async_copy(sbuf_v.at[pl.ds(0, NCH)],
                                agg_sh.at[pl.ds(t * TPT, NCH)], semS0).wait()
          return 0
        lax.fori_loop(0, TPT // NCH, zdrain, 0)

        def refix(n, _):
          sbuf_v[n, pl.ds(W - L, L)] = mix16
          return 0
        lax.fori_loop(0, NCH, refix, 0)

        def tfill(j, _):
          csrc_v[pl.ds(cnt + j * L, L)] = zi
          crt_v[pl.ds(cnt + j * L, L)] = zi
          cdst_v[pl.ds(cnt + j * L, L)] = dumps
          return 0
        lax.fori_loop(0, K // L, tfill, 0)
        plsc.subcore_barrier()

        nblk = (cnt + K - 1) // K

        def issue(b):
          @pl.when((b < nblk) & (b % 2 == 0))
          def _():
            pltpu.async_copy(ent_h.at[csrc_v.at[pl.ds(b * K, K)]],
                             gbuf_v.at[pl.ds(0, K)], sem0)

          @pl.when((b < nblk) & (b % 2 == 1))
          def _():
            pltpu.async_copy(ent_h.at[csrc_v.at[pl.ds(b * K, K)]],
                             gbuf_v.at[pl.ds(K, K)], sem1)

        def wait_scat(b):
          @pl.when((b >= 0) & (b % 2 == 0))
          def _():
            pltpu.make_async_copy(sbuf_v.at[pl.ds(0, K)],
                                  agg_sh.at[dst2_v.at[0]], semS0).wait()

          @pl.when((b >= 0) & (b % 2 == 1))
          def _():
            pltpu.make_async_copy(sbuf_v.at[pl.ds(K, K)],
                                  agg_sh.at[dst2_v.at[0]], semS1).wait()

        issue(0)

        def eblock(b, _):
          @pl.when(b < nblk)
          def _():
            def cpy(jj, _):
              dst2_v[b, pl.ds(jj * L, L)] = cdst_v[pl.ds(b * K + jj * L, L)]
              return 0
            lax.fori_loop(0, K // L, cpy, 0)

            @pl.when(b % 2 == 0)
            def _():
              pltpu.make_async_copy(ent_h.at[csrc_v.at[pl.ds(b * K, K)]],
                                    gbuf_v.at[pl.ds(0, K)], sem0).wait()

            @pl.when(b % 2 == 1)
            def _():
              pltpu.make_async_copy(ent_h.at[csrc_v.at[pl.ds(b * K, K)]],
                                    gbuf_v.at[pl.ds(K, K)], sem1).wait()

            issue(b + 1)
            wait_scat(b - 2)
            base = (b % 2) * K

            def emul(j, _):
              rt16 = crt_v[pl.ds(b * K + j * L, L)]
              for ee in range(L):
                e = j * L + ee
                wb = rt16[ee] * H
                for k in range(H // L):
                  sbuf_v[base + e, pl.ds(k * L, L)] = (
                      gbuf_v[base + e, pl.ds(k * L, L)]
                      * wrel_v[pl.ds(wb + k * L, L)])
              return 0
            lax.fori_loop(0, K // L, emul, 0)

            @pl.when(b % 2 == 0)
            def _():
              pltpu.async_copy(sbuf_v.at[pl.ds(0, K)],
                               agg_sh.at[dst2_v.at[b]], semS0, add=True)

            @pl.when(b % 2 == 1)
            def _():
              pltpu.async_copy(sbuf_v.at[pl.ds(K, K)],
                               agg_sh.at[dst2_v.at[b]], semS1, add=True)
          return 0
        lax.fori_loop(0, NB, eblock, 0)
        wait_scat(nblk - 2)
        wait_scat(nblk - 1)
        plsc.subcore_barrier()

        nchunks = TPT // NCH

        def issue_n(i):
          r = t * TPT + i * NCH

          @pl.when((i < nchunks) & (i % 2 == 0))
          def _():
            pltpu.async_copy(agg_sh.at[pl.ds(r, NCH)],
                             agg_v.at[pl.ds(0, NCH)], sem0)
            pltpu.async_copy(xself_h.at[pl.ds(lo + r, NCH)],
                             xs_v.at[pl.ds(0, NCH)], semS0)

          @pl.when((i < nchunks) & (i % 2 == 1))
          def _():
            pltpu.async_copy(agg_sh.at[pl.ds(r, NCH)],
                             agg_v.at[pl.ds(NCH, NCH)], sem1)
            pltpu.async_copy(xself_h.at[pl.ds(lo + r, NCH)],
                             xs_v.at[pl.ds(NCH, NCH)], semS1)

        issue_n(0)

        def nchunk(i, mx):
          r = t * TPT + i * NCH

          @pl.when(i % 2 == 0)
          def _():
            pltpu.make_async_copy(agg_sh.at[pl.ds(r, NCH)],
                                  agg_v.at[pl.ds(0, NCH)], sem0).wait()
            pltpu.make_async_copy(xself_h.at[pl.ds(lo + r, NCH)],
                                  xs_v.at[pl.ds(0, NCH)], semS0).wait()

          @pl.when(i % 2 == 1)
          def _():
            pltpu.make_async_copy(agg_sh.at[pl.ds(r, NCH)],
                                  agg_v.at[pl.ds(NCH, NCH)], sem1).wait()
            pltpu.make_async_copy(xself_h.at[pl.ds(lo + r, NCH)],
                                  xs_v.at[pl.ds(NCH, NCH)], semS1).wait()

          issue_n(i + 1)
          nb = (i % 2) * NCH

          def node(n, mx):
            dtail = agg_v[nb + n, pl.ds(W - L, L)]
            dinv = (1.0 / jnp.maximum(dtail, 1.0))[8]
            new = []
            for k in range(H // L):
              v = (agg_v[nb + n, pl.ds(k * L, L)] * dinv
                   + xs_v[nb + n, pl.ds(k * L, L)])
              v = jnp.maximum(v, 0.0)
              new.append(jnp.maximum(mx[k], v))
            return tuple(new)
          return lax.fori_loop(0, NCH, node, mx)

        mx = lax.fori_loop(0, nchunks, nchunk, mxall)
        return mx

      mx0 = tuple(jnp.zeros((L,), dtype=_f32) for _ in range(H // L))
      mx = lax.fori_loop(0, NSUB, subrange, mx0)
      for k in range(H // L):
        mymax_v[0, pl.ds(k * L, L)] = mx[k]
      pltpu.sync_copy(mymax_v, out_h.at[c, t, pl.ds(s, 1)])
      return 0

    lax.fori_loop(0, SEQ_LEN, snapshot, 0)

  return body(srcp, dstp, rtp, ent_pad, wrel_flat, xself)


def _tc_xself(ent_pad, W_self):
  blk = NPAD // 8

  def body(e_ref, w_ref, o_ref):
    o_ref[...] = jnp.dot(e_ref[...], w_ref[...],
                         preferred_element_type=_f32)

  return pl.pallas_call(
      body,
      grid=(8,),
      in_specs=[
          pl.BlockSpec((blk, H), lambda i: (i, 0)),
          pl.BlockSpec((H, H), lambda i: (0, 0)),
      ],
      out_specs=pl.BlockSpec((blk, H), lambda i: (i, 0)),
      out_shape=jax.ShapeDtypeStruct((NPAD, H), _f32),
  )(ent_pad, W_self)


def _tc_gru(gs2, W_ih, W_hh, b_ih, b_hh):
  def body(g_ref, wi_ref, wh_ref, bi_ref, bh_ref, o_ref):
    gs = jnp.max(g_ref[...], axis=(0, 1))
    bi = bi_ref[...]
    bh = bh_ref[...]

    h = jnp.zeros((1, H), dtype=_f32)
    for s in range(SEQ_LEN):
      x = gs[s:s + 1]
      gi = jnp.dot(x, wi_ref[...], preferred_element_type=_f32) + bi
      gh = jnp.dot(h, wh_ref[...], preferred_element_type=_f32) + bh
      ir, iz, in_ = gi[:, :H], gi[:, H:2 * H], gi[:, 2 * H:]
      hr, hz, hn = gh[:, :H], gh[:, H:2 * H], gh[:, 2 * H:]
      r = jax.nn.sigmoid(ir + hr)
      z = jax.nn.sigmoid(iz + hz)
      n = jnp.tanh(in_ + r * hn)
      h = (1.0 - z) * n + z * h
    o_ref[...] = h

  return pl.pallas_call(
      body,
      out_shape=jax.ShapeDtypeStruct((1, H), _f32),
  )(gs2, W_ih, W_hh, b_ih.reshape(1, 3 * H), b_hh.reshape(1, 3 * H))


LPAD = 50176
LBLK = 1024


def _tc_loss(s_q, W_lin_pad, b_lin_pad, tpo_pad):
  nt = LPAD // LBLK

  def body(sq_ref, w_ref, b_ref, t_ref, o_ref, acc):
    i = pl.program_id(0)

    @pl.when(i == 0)
    def _():
      acc[0] = -1e30
      acc[1] = 0.0
      acc[2] = 0.0

    p = (jnp.dot(sq_ref[...], w_ref[...], preferred_element_type=_f32)
         + b_ref[...])
    tb = jnp.sum(t_ref[...], axis=0, keepdims=True) * (1.0 / 64.0)
    m_old = acc[0]
    m_new = jnp.maximum(m_old, jnp.max(p))
    se = acc[1] * jnp.exp(m_old - m_new) + jnp.sum(jnp.exp(p - m_new))
    acc[0] = m_new
    acc[1] = se
    acc[2] = acc[2] + jnp.sum(tb * p)

    @pl.when(i == nt - 1)
    def _():
      o_ref[...] = jnp.full((1, 1), (acc[0] + jnp.log(acc[1])) - acc[2],
                            dtype=_f32)

  return pl.pallas_call(
      body,
      grid=(nt,),
      in_specs=[
          pl.BlockSpec((1, H), lambda i: (0, 0)),
          pl.BlockSpec((H, LBLK), lambda i: (0, i)),
          pl.BlockSpec((1, LBLK), lambda i: (0, i)),
          pl.BlockSpec((64, LBLK), lambda i: (0, i)),
      ],
      out_specs=pl.BlockSpec((1, 1), lambda i: (0, 0)),
      out_shape=jax.ShapeDtypeStruct((1, 1), _f32),
      scratch_shapes=[pltpu.SMEM((4,), _f32)],
  )(s_q, W_lin_pad, b_lin_pad, tpo_pad)


def kernel(t_list, true_prob_s, true_prob_o, edge_index, edge_type,
           ent_embeds, w_rel, W_self, W_ih, W_hh, b_ih, b_hh, W_lin, b_lin):
  del t_list, true_prob_s

  src = edge_index[0].reshape(SEQ_LEN, NS, EPT)
  dst = edge_index[1].reshape(SEQ_LEN, NS, EPT)
  rt = edge_type.reshape(SEQ_LEN, NS, EPT)
  pad = ((0, 0), (0, 0), (0, EPT_PAD - EPT))
  srcp = jnp.pad(src, pad).reshape(-1)
  dstp = jnp.pad(dst, pad, constant_values=-1).reshape(-1)
  rtp = jnp.pad(rt, pad).reshape(-1)

  ent_pad = jnp.pad(ent_embeds, ((0, NPAD - IN_DIM), (0, 0)))
  wrel_flat = w_rel.reshape(-1)

  xself = _tc_xself(ent_pad, W_self)
  gs2 = _sc_aggregate(srcp, dstp, rtp, ent_pad, wrel_flat, xself)
  s_q = _tc_gru(gs2, W_ih, W_hh, b_ih, b_hh)

  W_lin_pad = jnp.pad(W_lin, ((0, 0), (0, LPAD - IN_DIM)))
  b_lin_pad = jnp.pad(b_lin.reshape(1, IN_DIM), ((0, 0), (0, LPAD - IN_DIM)),
                      constant_values=-1e9)
  tpo_pad = jnp.pad(true_prob_o, ((0, 0), (0, LPAD - IN_DIM)))
  loss = _tc_loss(s_q, W_lin_pad, b_lin_pad, tpo_pad)
  return loss[0, 0]
